# baseline (speedup 1.0000x reference)
import functools

import jax
import jax.numpy as jnp
from jax import lax
from jax.experimental import pallas as pl
from jax.experimental.pallas import tpu as pltpu
from jax.experimental.pallas import tpu_sc as plsc

NC = 2
NS = 16
NW = NC * NS

V = 1000000
D = 64
FULL_BLOCKS = V // 128
BASE_BLOCKS = FULL_BLOCKS // NW
EXTRA_BLOCKS = FULL_BLOCKS - BASE_BLOCKS * NW
TAIL_ROWS = V - FULL_BLOCKS * 128
TAIL_WORKER = NW - 1

GROUP = 128
NBUF = 4


def _row_iotas():
    base = lax.iota(jnp.int32, 16)
    return [base + 16 * g for g in range(8)]


def _transpose_pairs(sbuf, tbuf, width):
    rowv = _row_iotas()

    def oct_body(i, carry):
        m0 = i * 8
        for dm in range(8):
            m = m0 + dm
            c0v = jnp.full((16,), 2 * m, jnp.int32)
            c1v = c0v + 1
            for g in range(8):
                colv = c0v if g < 4 else c1v
                vals = plsc.load_gather(sbuf, [rowv[g % 4], colv])
                tbuf[m, pl.ds(16 * g, 16)] = vals
        return carry

    lax.fori_loop(0, width // 16, oct_body, 0)


@functools.partial(jax.jit, static_argnames=())
def _convert(wt, tail2d):
    mesh = plsc.VectorSubcoreMesh(core_axis_name="c", subcore_axis_name="s")

    @functools.partial(
        pl.kernel,
        out_type=jax.ShapeDtypeStruct((V // 2, 128), jnp.float32),
        mesh=mesh,
        scratch_types=[
            [pltpu.VMEM((D, 128), jnp.float32) for _ in range(2)],
            [pltpu.VMEM((D, 128), jnp.float32) for _ in range(2)],
            [pltpu.SemaphoreType.DMA for _ in range(2)],
            [pltpu.SemaphoreType.DMA for _ in range(2)],
        ],
        compiler_params=pltpu.CompilerParams(use_tc_tiling_on_sc=True, needs_layout_passes=False),
    )
    def body(wt_hbm, tail_hbm, out_hbm, sbufs, tbufs, gsems, wsems):
        wid = lax.axis_index("s") * NC + lax.axis_index("c")
        base_blk = wid * BASE_BLOCKS

        def do_block(blk, sbuf, tbuf, gsem, wsem, first):
            @pl.when(jnp.logical_not(first))
            def _():
                pltpu.make_async_copy(
                    tbuf, out_hbm.at[pl.ds(blk * 64, 64)], wsem
                ).wait()

            pltpu.async_copy(
                wt_hbm.at[:, pl.ds(blk * 128, 128)], sbuf, gsem
            ).wait()
            _transpose_pairs(sbuf, tbuf, 128)
            pltpu.async_copy(tbuf, out_hbm.at[pl.ds(blk * 64, 64)], wsem)

        def pair_body(i, carry):
            for b in range(2):
                do_block(base_blk + 2 * i + b, sbufs[b], tbufs[b],
                         gsems[b], wsems[b], i == 0)
            return carry

        lax.fori_loop(0, BASE_BLOCKS // 2, pair_body, 0)
        for b in range(2):
            pltpu.make_async_copy(
                tbufs[b], out_hbm.at[pl.ds(0, 64)], wsems[b]
            ).wait()

        @pl.when(wid < EXTRA_BLOCKS)
        def _():
            blk = FULL_BLOCKS - EXTRA_BLOCKS + wid
            pltpu.async_copy(
                wt_hbm.at[:, pl.ds(blk * 128, 128)], sbufs[0], gsems[0]
            ).wait()
            _transpose_pairs(sbufs[0], tbufs[0], 128)
            pltpu.async_copy(
                tbufs[0], out_hbm.at[pl.ds(blk * 64, 64)], wsems[0]
            ).wait()

        @pl.when(wid == TAIL_WORKER)
        def _():
            pltpu.sync_copy(tail_hbm, tbufs[1].at[pl.ds(0, TAIL_ROWS // 2)])
            pltpu.sync_copy(
                tbufs[1].at[pl.ds(0, TAIL_ROWS // 2)],
                out_hbm.at[pl.ds(FULL_BLOCKS * 64, TAIL_ROWS // 2)],
            )

    return body(wt, tail2d)


@functools.partial(jax.jit, static_argnames=())
def _gather(y2, table):
    n_units = y2.shape[0]
    units_per_w = n_units // NW
    bt_per_t = 16384 // GROUP
    mesh = plsc.VectorSubcoreMesh(core_axis_name="c", subcore_axis_name="s")

    @functools.partial(
        pl.kernel,
        out_type=jax.ShapeDtypeStruct((50, 8, bt_per_t, 8, GROUP),
                                      jnp.float32),
        mesh=mesh,
        scratch_types=[
            pltpu.VMEM((units_per_w, GROUP), jnp.int32),
            [pltpu.VMEM((GROUP, D), jnp.float32) for _ in range(NBUF)],
            [pltpu.VMEM((8, 8, GROUP), jnp.float32) for _ in range(NBUF)],
            [pltpu.SemaphoreType.DMA for _ in range(NBUF)],
            [pltpu.SemaphoreType.DMA for _ in range(NBUF)],
        ],
        compiler_params=pltpu.CompilerParams(use_tc_tiling_on_sc=False, needs_layout_passes=False),
    )
    def body(y_hbm, w_hbm, out_hbm, idx_v, rows, tbs, gsems, wsems):
        wid = lax.axis_index("s") * NC + lax.axis_index("c")
        ubase = wid * units_per_w
        pltpu.sync_copy(
            y_hbm.at[pl.ds(pl.multiple_of(ubase, 8), units_per_w)], idx_v
        )
        rowv = _row_iotas()

        def transpose_rows(rbuf, tb):
            def ft_body(ft, carry):
                base = jnp.full((16,), ft * 8, jnp.int32)
                for fi in range(8):
                    colv = base + fi
                    for g in range(8):
                        vals = plsc.load_gather(rbuf, [rowv[g], colv])
                        tb[ft, fi, pl.ds(16 * g, 16)] = vals
                return carry

            lax.fori_loop(0, 8, ft_body, 0)

        def unit_out_ref(u):
            t = u // bt_per_t
            bt = lax.rem(u, bt_per_t)
            return out_hbm.at[t, :, bt, :, :]

        def ring_body(i, carry):
            for b in range(NBUF):
                u = ubase + i * NBUF + b

                @pl.when(i > 0)
                def _():
                    pltpu.make_async_copy(tbs[b], unit_out_ref(u),
                                          wsems[b]).wait()

                pltpu.async_copy(
                    w_hbm.at[idx_v.at[i * NBUF + b]], rows[b], gsems[b]
                )
            for b in range(NBUF):
                u = ubase + i * NBUF + b
                pltpu.make_async_copy(
                    w_hbm.at[idx_v.at[i * NBUF + b]], rows[b], gsems[b]
                ).wait()
                transpose_rows(rows[b], tbs[b])
                pltpu.async_copy(tbs[b], unit_out_ref(u), wsems[b])
            return carry

        lax.fori_loop(0, units_per_w // NBUF, ring_body, 0)
        for b in range(NBUF):
            pltpu.make_async_copy(tbs[b], unit_out_ref(ubase + b),
                                  wsems[b]).wait()

    return body(y2, table)


def kernel(Y, weight):
    wt = weight.T
    tail2d = weight[FULL_BLOCKS * 128:].reshape(TAIL_ROWS // 2, 128)
    table2 = _convert(wt, tail2d)
    table = table2.reshape(V, D)
    y2 = Y.astype(jnp.int32).T.reshape(6400, GROUP)
    out5 = _gather(y2, table)
    return out5.transpose(2, 4, 0, 1, 3).reshape(16384, 50, D)


# speedup vs baseline: 1.5610x; 1.5610x over previous
import functools

import jax
import jax.numpy as jnp
from jax import lax
from jax.experimental import pallas as pl
from jax.experimental.pallas import tpu as pltpu
from jax.experimental.pallas import tpu_sc as plsc

NC = 2
NS = 16
NW = NC * NS

V = 1000000
D = 64
FULL_BLOCKS = V // 128
BASE_BLOCKS = FULL_BLOCKS // NW
EXTRA_BLOCKS = FULL_BLOCKS - BASE_BLOCKS * NW
TAIL_ROWS = V - FULL_BLOCKS * 128
TAIL_WORKER = NW - 1

GROUP = 128
NBUF = 4


def _row_iotas():
    base = lax.iota(jnp.int32, 16)
    return [base + 16 * g for g in range(8)]


def _transpose_pairs(sbuf, tbuf, width):
    rowv = _row_iotas()

    @plsc.parallel_loop(0, width // 2, unroll=8)
    def m_body(m):
        c0v = jnp.full((16,), 2 * m, jnp.int32)
        c1v = c0v + 1
        for g in range(8):
            colv = c0v if g < 4 else c1v
            vals = plsc.load_gather(sbuf, [rowv[g % 4], colv])
            tbuf[m, pl.ds(16 * g, 16)] = vals


@functools.partial(jax.jit, static_argnames=())
def _convert(wt, tail2d):
    mesh = plsc.VectorSubcoreMesh(core_axis_name="c", subcore_axis_name="s")

    @functools.partial(
        pl.kernel,
        out_type=jax.ShapeDtypeStruct((V // 2, 128), jnp.float32),
        mesh=mesh,
        scratch_types=[
            [pltpu.VMEM((D, 128), jnp.float32) for _ in range(2)],
            [pltpu.VMEM((D, 128), jnp.float32) for _ in range(2)],
            [pltpu.SemaphoreType.DMA for _ in range(2)],
            [pltpu.SemaphoreType.DMA for _ in range(2)],
        ],
        compiler_params=pltpu.CompilerParams(use_tc_tiling_on_sc=True, needs_layout_passes=False),
    )
    def body(wt_hbm, tail_hbm, out_hbm, sbufs, tbufs, gsems, wsems):
        wid = lax.axis_index("s") * NC + lax.axis_index("c")
        base_blk = wid * BASE_BLOCKS

        def do_block(blk, sbuf, tbuf, gsem, wsem, first):
            @pl.when(jnp.logical_not(first))
            def _():
                pltpu.make_async_copy(
                    tbuf, out_hbm.at[pl.ds(blk * 64, 64)], wsem
                ).wait()

            pltpu.async_copy(
                wt_hbm.at[:, pl.ds(blk * 128, 128)], sbuf, gsem
            ).wait()
            _transpose_pairs(sbuf, tbuf, 128)
            pltpu.async_copy(tbuf, out_hbm.at[pl.ds(blk * 64, 64)], wsem)

        def pair_body(i, carry):
            for b in range(2):
                do_block(base_blk + 2 * i + b, sbufs[b], tbufs[b],
                         gsems[b], wsems[b], i == 0)
            return carry

        lax.fori_loop(0, BASE_BLOCKS // 2, pair_body, 0)
        for b in range(2):
            pltpu.make_async_copy(
                tbufs[b], out_hbm.at[pl.ds(0, 64)], wsems[b]
            ).wait()

        @pl.when(wid < EXTRA_BLOCKS)
        def _():
            blk = FULL_BLOCKS - EXTRA_BLOCKS + wid
            pltpu.async_copy(
                wt_hbm.at[:, pl.ds(blk * 128, 128)], sbufs[0], gsems[0]
            ).wait()
            _transpose_pairs(sbufs[0], tbufs[0], 128)
            pltpu.async_copy(
                tbufs[0], out_hbm.at[pl.ds(blk * 64, 64)], wsems[0]
            ).wait()

        @pl.when(wid == TAIL_WORKER)
        def _():
            pltpu.sync_copy(tail_hbm, tbufs[1].at[pl.ds(0, TAIL_ROWS // 2)])
            pltpu.sync_copy(
                tbufs[1].at[pl.ds(0, TAIL_ROWS // 2)],
                out_hbm.at[pl.ds(FULL_BLOCKS * 64, TAIL_ROWS // 2)],
            )

    return body(wt, tail2d)


@functools.partial(jax.jit, static_argnames=())
def _gather(y2, table):
    n_units = y2.shape[0]
    units_per_w = n_units // NW
    bt_per_t = 16384 // GROUP
    mesh = plsc.VectorSubcoreMesh(core_axis_name="c", subcore_axis_name="s")

    @functools.partial(
        pl.kernel,
        out_type=jax.ShapeDtypeStruct((50, 8, bt_per_t, 8, GROUP),
                                      jnp.float32),
        mesh=mesh,
        scratch_types=[
            pltpu.VMEM((units_per_w, GROUP), jnp.int32),
            [pltpu.VMEM((GROUP, D), jnp.float32) for _ in range(NBUF)],
            [pltpu.VMEM((8, 8, GROUP), jnp.float32) for _ in range(NBUF)],
            [pltpu.SemaphoreType.DMA for _ in range(NBUF)],
            [pltpu.SemaphoreType.DMA for _ in range(NBUF)],
        ],
        compiler_params=pltpu.CompilerParams(use_tc_tiling_on_sc=False, needs_layout_passes=False),
    )
    def body(y_hbm, w_hbm, out_hbm, idx_v, rows, tbs, gsems, wsems):
        wid = lax.axis_index("s") * NC + lax.axis_index("c")
        ubase = wid * units_per_w
        pltpu.sync_copy(
            y_hbm.at[pl.ds(pl.multiple_of(ubase, 8), units_per_w)], idx_v
        )
        rowv = _row_iotas()

        def transpose_rows(rbuf, tb):
            @plsc.parallel_loop(0, 8, unroll=2)
            def ft_body(ft):
                base = jnp.full((16,), ft * 8, jnp.int32)
                for fi in range(8):
                    colv = base + fi
                    for g in range(8):
                        vals = plsc.load_gather(rbuf, [rowv[g], colv])
                        tb[ft, fi, pl.ds(16 * g, 16)] = vals

        def unit_out_ref(u):
            t = u // bt_per_t
            bt = lax.rem(u, bt_per_t)
            return out_hbm.at[t, :, bt, :, :]

        def ring_body(i, carry):
            for b in range(NBUF):
                u = ubase + i * NBUF + b

                @pl.when(i > 0)
                def _():
                    pltpu.make_async_copy(tbs[b], unit_out_ref(u),
                                          wsems[b]).wait()

                pltpu.async_copy(
                    w_hbm.at[idx_v.at[i * NBUF + b]], rows[b], gsems[b]
                )
            for b in range(NBUF):
                u = ubase + i * NBUF + b
                pltpu.make_async_copy(
                    w_hbm.at[idx_v.at[i * NBUF + b]], rows[b], gsems[b]
                ).wait()
                transpose_rows(rows[b], tbs[b])
                pltpu.async_copy(tbs[b], unit_out_ref(u), wsems[b])
            return carry

        lax.fori_loop(0, units_per_w // NBUF, ring_body, 0)
        for b in range(NBUF):
            pltpu.make_async_copy(tbs[b], unit_out_ref(ubase + b),
                                  wsems[b]).wait()

    return body(y2, table)


def kernel(Y, weight):
    wt = weight.T
    tail2d = weight[FULL_BLOCKS * 128:].reshape(TAIL_ROWS // 2, 128)
    table2 = _convert(wt, tail2d)
    table = table2.reshape(V, D)
    y2 = Y.astype(jnp.int32).T.reshape(6400, GROUP)
    out5 = _gather(y2, table)
    return out5.transpose(2, 4, 0, 1, 3).reshape(16384, 50, D)


# speedup vs baseline: 2.3439x; 1.5016x over previous
import functools

import jax
import jax.numpy as jnp
from jax import lax
from jax.experimental import pallas as pl
from jax.experimental.pallas import tpu as pltpu
from jax.experimental.pallas import tpu_sc as plsc

NC = 2
NS = 16
NW = NC * NS

V = 1000000
D = 64
FULL_BLOCKS = V // 128
BASE_BLOCKS = FULL_BLOCKS // NW
EXTRA_BLOCKS = FULL_BLOCKS - BASE_BLOCKS * NW
TAIL_ROWS = V - FULL_BLOCKS * 128
TAIL_WORKER = NW - 1

GROUP = 128
NBUF = 4


PAD = 129


def _transpose_pairs(sbuf, tbuf, width):
    iota = lax.iota(jnp.int32, 16)
    mv = [(iota // 2) + 8 * k for k in range(width // 16)]
    qb = (iota % 2) * 64

    @plsc.parallel_loop(0, D, unroll=2)
    def f_body(f):
        qv = qb + f
        for k in range(width // 16):
            vals = sbuf[f, pl.ds(16 * k, 16)]
            plsc.store_scatter(tbuf, [mv[k], qv], vals)


@functools.partial(jax.jit, static_argnames=())
def _convert(wt, tail2d):
    mesh = plsc.VectorSubcoreMesh(core_axis_name="c", subcore_axis_name="s")

    @functools.partial(
        pl.kernel,
        out_type=jax.ShapeDtypeStruct((V // 2, 128), jnp.float32),
        mesh=mesh,
        scratch_types=[
            [pltpu.VMEM((D, 128), jnp.float32) for _ in range(2)],
            [pltpu.VMEM((D, PAD), jnp.float32) for _ in range(2)],
            [pltpu.SemaphoreType.DMA for _ in range(2)],
            [pltpu.SemaphoreType.DMA for _ in range(2)],
        ],
        compiler_params=pltpu.CompilerParams(use_tc_tiling_on_sc=True, needs_layout_passes=False),
    )
    def body(wt_hbm, tail_hbm, out_hbm, sbufs, tbufs, gsems, wsems):
        wid = lax.axis_index("s") * NC + lax.axis_index("c")
        base_blk = wid * BASE_BLOCKS

        def do_block(blk, sbuf, tbuf, gsem, wsem, first):
            tsrc = tbuf.at[:, pl.ds(0, 128)]

            @pl.when(jnp.logical_not(first))
            def _():
                pltpu.make_async_copy(
                    tsrc, out_hbm.at[pl.ds(blk * 64, 64)], wsem
                ).wait()

            pltpu.async_copy(
                wt_hbm.at[:, pl.ds(blk * 128, 128)], sbuf, gsem
            ).wait()
            _transpose_pairs(sbuf, tbuf, 128)
            pltpu.async_copy(tsrc, out_hbm.at[pl.ds(blk * 64, 64)], wsem)

        def pair_body(i, carry):
            for b in range(2):
                do_block(base_blk + 2 * i + b, sbufs[b], tbufs[b],
                         gsems[b], wsems[b], i == 0)
            return carry

        lax.fori_loop(0, BASE_BLOCKS // 2, pair_body, 0)
        for b in range(2):
            pltpu.make_async_copy(
                tbufs[b].at[:, pl.ds(0, 128)], out_hbm.at[pl.ds(0, 64)],
                wsems[b]
            ).wait()

        @pl.when(wid < EXTRA_BLOCKS)
        def _():
            blk = FULL_BLOCKS - EXTRA_BLOCKS + wid
            pltpu.async_copy(
                wt_hbm.at[:, pl.ds(blk * 128, 128)], sbufs[0], gsems[0]
            ).wait()
            _transpose_pairs(sbufs[0], tbufs[0], 128)
            pltpu.async_copy(
                tbufs[0].at[:, pl.ds(0, 128)],
                out_hbm.at[pl.ds(blk * 64, 64)], wsems[0]
            ).wait()

        @pl.when(wid == TAIL_WORKER)
        def _():
            tsl = tbufs[1].at[pl.ds(0, TAIL_ROWS // 2), pl.ds(0, 128)]
            pltpu.sync_copy(tail_hbm, tsl)
            pltpu.sync_copy(
                tsl, out_hbm.at[pl.ds(FULL_BLOCKS * 64, TAIL_ROWS // 2)]
            )

    return body(wt, tail2d)


@functools.partial(jax.jit, static_argnames=())
def _gather(y2, table):
    n_units = y2.shape[0]
    units_per_w = n_units // NW
    bt_per_t = 16384 // GROUP
    mesh = plsc.VectorSubcoreMesh(core_axis_name="c", subcore_axis_name="s")

    @functools.partial(
        pl.kernel,
        out_type=jax.ShapeDtypeStruct((50, 8, bt_per_t, 8, GROUP),
                                      jnp.float32),
        mesh=mesh,
        scratch_types=[
            pltpu.VMEM((units_per_w, GROUP), jnp.int32),
            [pltpu.VMEM((GROUP, D), jnp.float32) for _ in range(NBUF)],
            [pltpu.VMEM((8, 8, PAD), jnp.float32) for _ in range(NBUF)],
            [pltpu.SemaphoreType.DMA for _ in range(NBUF)],
            [pltpu.SemaphoreType.DMA for _ in range(NBUF)],
        ],
        compiler_params=pltpu.CompilerParams(use_tc_tiling_on_sc=False, needs_layout_passes=False),
    )
    def body(y_hbm, w_hbm, out_hbm, idx_v, rows, tbs, gsems, wsems):
        wid = lax.axis_index("s") * NC + lax.axis_index("c")
        ubase = wid * units_per_w
        pltpu.sync_copy(
            y_hbm.at[pl.ds(pl.multiple_of(ubase, 8), units_per_w)], idx_v
        )
        iota = lax.iota(jnp.int32, 16)
        ftv = [(iota + 16 * k) // 8 for k in range(4)]
        fiv = [(iota + 16 * k) % 8 for k in range(4)]

        def transpose_rows(rbuf, tb):
            @plsc.parallel_loop(0, GROUP, unroll=4)
            def bi_body(bi):
                colv = jnp.full((16,), bi, jnp.int32)
                for k in range(4):
                    vals = rbuf[bi, pl.ds(16 * k, 16)]
                    plsc.store_scatter(tb, [ftv[k], fiv[k], colv], vals)

        def unit_out_ref(u):
            t = u // bt_per_t
            bt = lax.rem(u, bt_per_t)
            return out_hbm.at[t, :, bt, :, :]

        def ring_body(i, carry):
            for b in range(NBUF):
                u = ubase + i * NBUF + b

                @pl.when(i > 0)
                def _():
                    pltpu.make_async_copy(tbs[b].at[:, :, pl.ds(0, GROUP)],
                                          unit_out_ref(u), wsems[b]).wait()

                pltpu.async_copy(
                    w_hbm.at[idx_v.at[i * NBUF + b]], rows[b], gsems[b]
                )
            for b in range(NBUF):
                u = ubase + i * NBUF + b
                pltpu.make_async_copy(
                    w_hbm.at[idx_v.at[i * NBUF + b]], rows[b], gsems[b]
                ).wait()
                transpose_rows(rows[b], tbs[b])
                pltpu.async_copy(tbs[b].at[:, :, pl.ds(0, GROUP)],
                                 unit_out_ref(u), wsems[b])
            return carry

        lax.fori_loop(0, units_per_w // NBUF, ring_body, 0)
        for b in range(NBUF):
            pltpu.make_async_copy(tbs[b].at[:, :, pl.ds(0, GROUP)],
                                  unit_out_ref(ubase + b), wsems[b]).wait()

    return body(y2, table)


def kernel(Y, weight):
    wt = weight.T
    tail2d = weight[FULL_BLOCKS * 128:].reshape(TAIL_ROWS // 2, 128)
    table2 = _convert(wt, tail2d)
    table = table2.reshape(V, D)
    y2 = Y.astype(jnp.int32).T.reshape(6400, GROUP)
    out5 = _gather(y2, table)
    return out5.transpose(2, 4, 0, 1, 3).reshape(16384, 50, D)


# speedup vs baseline: 2.9027x; 1.2384x over previous
import functools

import jax
import jax.numpy as jnp
from jax import lax
from jax.experimental import pallas as pl
from jax.experimental.pallas import tpu as pltpu
from jax.experimental.pallas import tpu_sc as plsc

NC = 2
NS = 16
NW = NC * NS

V = 1000000
D = 64
FULL_BLOCKS = V // 128
BASE_BLOCKS = FULL_BLOCKS // NW
EXTRA_BLOCKS = FULL_BLOCKS - BASE_BLOCKS * NW
TAIL_ROWS = V - FULL_BLOCKS * 128
TAIL_WORKER = NW - 1

GROUP = 128
NBUF = 4
NB1 = 4


PAD = 129


def _transpose_pairs(sbuf, tbuf, width):
    iota = lax.iota(jnp.int32, 16)
    mv = [(iota // 2) + 8 * k for k in range(width // 16)]
    qb = (iota % 2) * 64

    @plsc.parallel_loop(0, D, unroll=2)
    def f_body(f):
        qv = qb + f
        for k in range(width // 16):
            vals = sbuf[f, pl.ds(16 * k, 16)]
            plsc.store_scatter(tbuf, [mv[k], qv], vals)


@functools.partial(jax.jit, static_argnames=())
def _convert(wt, tail2d):
    mesh = plsc.VectorSubcoreMesh(core_axis_name="c", subcore_axis_name="s")

    @functools.partial(
        pl.kernel,
        out_type=jax.ShapeDtypeStruct((V // 2, 128), jnp.float32),
        mesh=mesh,
        scratch_types=[
            [pltpu.VMEM((D, 128), jnp.float32) for _ in range(NB1)],
            [pltpu.VMEM((D, PAD), jnp.float32) for _ in range(NB1)],
            [pltpu.SemaphoreType.DMA for _ in range(NB1)],
            [pltpu.SemaphoreType.DMA for _ in range(NB1)],
        ],
        compiler_params=pltpu.CompilerParams(use_tc_tiling_on_sc=True, needs_layout_passes=False),
    )
    def body(wt_hbm, tail_hbm, out_hbm, sbufs, tbufs, gsems, wsems):
        wid = lax.axis_index("s") * NC + lax.axis_index("c")
        base_blk = wid * BASE_BLOCKS

        def in_copy(blk, b):
            return pltpu.make_async_copy(
                wt_hbm.at[:, pl.ds(blk * 128, 128)], sbufs[b], gsems[b]
            )

        for b in range(NB1):
            in_copy(base_blk + b, b).start()

        def ring(i, carry):
            for b in range(NB1):
                c = i * NB1 + b
                blk = base_blk + c
                in_copy(blk, b).wait()

                @pl.when(i > 0)
                def _():
                    pltpu.make_async_copy(
                        tbufs[b].at[:, pl.ds(0, 128)],
                        out_hbm.at[pl.ds(blk * 64, 64)], wsems[b]
                    ).wait()

                _transpose_pairs(sbufs[b], tbufs[b], 128)
                pltpu.async_copy(
                    tbufs[b].at[:, pl.ds(0, 128)],
                    out_hbm.at[pl.ds(blk * 64, 64)], wsems[b]
                )

                @pl.when(c + NB1 < BASE_BLOCKS)
                def _():
                    in_copy(blk + NB1, b).start()
            return carry

        lax.fori_loop(0, BASE_BLOCKS // NB1, ring, 0)
        for b in range(NB1):
            pltpu.make_async_copy(
                tbufs[b].at[:, pl.ds(0, 128)], out_hbm.at[pl.ds(0, 64)],
                wsems[b]
            ).wait()

        @pl.when(wid < EXTRA_BLOCKS)
        def _():
            blk = FULL_BLOCKS - EXTRA_BLOCKS + wid
            in_copy(blk, 0).start()
            in_copy(blk, 0).wait()
            _transpose_pairs(sbufs[0], tbufs[0], 128)
            pltpu.async_copy(
                tbufs[0].at[:, pl.ds(0, 128)],
                out_hbm.at[pl.ds(blk * 64, 64)], wsems[0]
            ).wait()

        @pl.when(wid == TAIL_WORKER)
        def _():
            tsl = tbufs[1].at[pl.ds(0, TAIL_ROWS // 2), pl.ds(0, 128)]
            pltpu.sync_copy(tail_hbm, tsl)
            pltpu.sync_copy(
                tsl, out_hbm.at[pl.ds(FULL_BLOCKS * 64, TAIL_ROWS // 2)]
            )

    return body(wt, tail2d)


@functools.partial(jax.jit, static_argnames=())
def _gather(y2, table):
    n_units = y2.shape[0]
    units_per_w = n_units // NW
    bt_per_t = 16384 // GROUP
    mesh = plsc.VectorSubcoreMesh(core_axis_name="c", subcore_axis_name="s")

    @functools.partial(
        pl.kernel,
        out_type=jax.ShapeDtypeStruct((50, 8, bt_per_t, 8, GROUP),
                                      jnp.float32),
        mesh=mesh,
        scratch_types=[
            pltpu.VMEM((units_per_w, GROUP), jnp.int32),
            [pltpu.VMEM((GROUP, D), jnp.float32) for _ in range(NBUF)],
            [pltpu.VMEM((8, 8, PAD), jnp.float32) for _ in range(NBUF)],
            [pltpu.SemaphoreType.DMA for _ in range(NBUF)],
            [pltpu.SemaphoreType.DMA for _ in range(NBUF)],
        ],
        compiler_params=pltpu.CompilerParams(use_tc_tiling_on_sc=False, needs_layout_passes=False),
    )
    def body(y_hbm, w_hbm, out_hbm, idx_v, rows, tbs, gsems, wsems):
        wid = lax.axis_index("s") * NC + lax.axis_index("c")
        ubase = wid * units_per_w
        pltpu.sync_copy(
            y_hbm.at[pl.ds(pl.multiple_of(ubase, 8), units_per_w)], idx_v
        )
        iota = lax.iota(jnp.int32, 16)
        ftv = [(iota + 16 * k) // 8 for k in range(4)]
        fiv = [(iota + 16 * k) % 8 for k in range(4)]

        def transpose_rows(rbuf, tb):
            @plsc.parallel_loop(0, GROUP, unroll=4)
            def bi_body(bi):
                colv = jnp.full((16,), bi, jnp.int32)
                for k in range(4):
                    vals = rbuf[bi, pl.ds(16 * k, 16)]
                    plsc.store_scatter(tb, [ftv[k], fiv[k], colv], vals)

        def unit_out_ref(u):
            t = u // bt_per_t
            bt = lax.rem(u, bt_per_t)
            return out_hbm.at[t, :, bt, :, :]

        def ring_body(i, carry):
            for b in range(NBUF):
                u = ubase + i * NBUF + b

                @pl.when(i > 0)
                def _():
                    pltpu.make_async_copy(tbs[b].at[:, :, pl.ds(0, GROUP)],
                                          unit_out_ref(u), wsems[b]).wait()

                pltpu.async_copy(
                    w_hbm.at[idx_v.at[i * NBUF + b]], rows[b], gsems[b]
                )
            for b in range(NBUF):
                u = ubase + i * NBUF + b
                pltpu.make_async_copy(
                    w_hbm.at[idx_v.at[i * NBUF + b]], rows[b], gsems[b]
                ).wait()
                transpose_rows(rows[b], tbs[b])
                pltpu.async_copy(tbs[b].at[:, :, pl.ds(0, GROUP)],
                                 unit_out_ref(u), wsems[b])
            return carry

        lax.fori_loop(0, units_per_w // NBUF, ring_body, 0)
        for b in range(NBUF):
            pltpu.make_async_copy(tbs[b].at[:, :, pl.ds(0, GROUP)],
                                  unit_out_ref(ubase + b), wsems[b]).wait()

    return body(y2, table)


def kernel(Y, weight):
    wt = weight.T
    tail2d = weight[FULL_BLOCKS * 128:].reshape(TAIL_ROWS // 2, 128)
    table2 = _convert(wt, tail2d)
    table = table2.reshape(V, D)
    y2 = Y.astype(jnp.int32).T.reshape(6400, GROUP)
    out5 = _gather(y2, table)
    return out5.transpose(2, 4, 0, 1, 3).reshape(16384, 50, D)
